# baseline (speedup 1.0000x reference)
import functools

import jax
import jax.numpy as jnp
from jax import lax
from jax.experimental import pallas as pl
from jax.experimental.pallas import tpu as pltpu

F32 = jnp.float32
_NSEG = 512
_W = 32
_CHUNK = 128
_EPS = 1e-5


def _mm(a, b):
    return jnp.dot(a, b, preferred_element_type=F32)


def _mmT(a, b):
    return lax.dot_general(a, b, (((0,), (0,)), ((), ())),
                           preferred_element_type=F32)


def _mlp(h, w1, b1, w2, b2):
    t = jnp.maximum(_mm(h, w1) + b1, 0.0)
    return _mm(t, w2) + b2


def _onehot(rel, w, dtype):
    b_tok = rel.shape[0]
    return (rel[:, None] == lax.broadcasted_iota(
        jnp.int32, (b_tok, w), 1)).astype(dtype)


def _block_window(ids, nseg):
    b_tok = ids.shape[0]
    base = (jnp.minimum(ids[0], nseg - _W) // 8) * 8
    ovf = (ids[b_tok - 1] - base) >= _W
    return base, ovf


def _bn_affine(n_tok, ssh, ssp, cnt, sq0, rw1, rb1, rw2, rb2, g, bta):
    rho = _mlp(ssh, rw1, rb1, rw2, rb2)
    s1 = jnp.sum(ssp, axis=0) + jnp.sum(cnt * rho, axis=0)
    s2 = (sq0 + 2.0 * jnp.sum(ssp * rho, axis=0)
          + jnp.sum(cnt * rho * rho, axis=0))
    m = s1 / n_tok
    v = s2 / n_tok - m * m
    a = g * lax.rsqrt(v + _EPS)
    return rho, a, bta - a * m


def _gather_rho_slow(ids, nseg, rho_ref, shape):
    acc = jnp.zeros(shape, F32)
    for s0 in range(0, nseg, _CHUNK):
        acc = acc + _mm(_onehot(ids - s0, _CHUNK, F32),
                        rho_ref[s0:s0 + _CHUNK, :])
    return acc


def _passA_kernel(nseg, ids_ref, h_ref, w1_ref, b1_ref, w2_ref, b2_ref,
                  xphi_ref, ssh_ref, ssp_ref, cnt_ref, sq_ref):
    i = pl.program_id(0)

    @pl.when(i == 0)
    def _init():
        ssh_ref[...] = jnp.zeros_like(ssh_ref)
        ssp_ref[...] = jnp.zeros_like(ssp_ref)
        cnt_ref[...] = jnp.zeros_like(cnt_ref)
        sq_ref[...] = jnp.zeros_like(sq_ref)

    h = h_ref[...]
    xphi = _mlp(h, w1_ref[...], b1_ref[...], w2_ref[...], b2_ref[...])
    xphi_ref[...] = xphi

    sq = jnp.sum(xphi * xphi, axis=0, keepdims=True)
    sq_ref[...] += jnp.broadcast_to(sq, sq_ref.shape)

    ids = ids_ref[0, 0, :]
    base, ovf = _block_window(ids, nseg)

    @pl.when(jnp.logical_not(ovf))
    def _fast():
        oh = _onehot(ids - base, _W, F32)
        ssh_ref[pl.ds(base, _W), :] += _mmT(oh, h)
        ssp_ref[pl.ds(base, _W), :] += _mmT(oh, xphi)
        c = jnp.sum(oh, axis=0, keepdims=True)
        cnt_ref[pl.ds(base, _W), :] += jnp.broadcast_to(
            c.T, (_W, cnt_ref.shape[1]))

    @pl.when(ovf)
    def _slow():
        for s0 in range(0, nseg, _CHUNK):
            ohc = _onehot(ids - s0, _CHUNK, F32)
            ssh_ref[s0:s0 + _CHUNK, :] += _mmT(ohc, h)
            ssp_ref[s0:s0 + _CHUNK, :] += _mmT(ohc, xphi)
            c = jnp.sum(ohc, axis=0, keepdims=True)
            cnt_ref[s0:s0 + _CHUNK, :] += jnp.broadcast_to(
                c.T, (_CHUNK, cnt_ref.shape[1]))


def _fusedBA_kernel(nseg, n_tok, store_next, recompute,
                    ids_ref, h_ref, *rest):
    rest = list(rest)
    fw = None
    if recompute:
        fw = [rest.pop(0) for _ in range(4)]
    else:
        xphi_ref = rest.pop(0)
    (ssh_ref, ssp_ref, cnt_ref, sq_ref,
     rw1_ref, rb1_ref, rw2_ref, rb2_ref, g_ref, bta_ref,
     pw1_ref, pb1_ref, pw2_ref, pb2_ref, hout_ref) = rest[:15]
    rest = rest[15:]
    if store_next:
        xphi2_ref = rest.pop(0)
    (ssh2_ref, ssp2_ref, sq2_ref, rho_ref, ab_ref) = rest

    i = pl.program_id(0)

    @pl.when(i == 0)
    def _prep():
        cnt = cnt_ref[...]
        rho, a, b = _bn_affine(
            n_tok, ssh_ref[...], ssp_ref[...], cnt, sq_ref[0, :],
            rw1_ref[...], rb1_ref[...], rw2_ref[...], rb2_ref[...],
            g_ref[0, :], bta_ref[0, :])
        rho_ref[...] = rho
        ab_ref[0, :] = a
        ab_ref[1, :] = b
        ssh2_ref[...] = (ssh_ref[...] + a * (ssp_ref[...] + cnt * rho)
                         + cnt * b)
        ssp2_ref[...] = jnp.zeros_like(ssp2_ref)
        sq2_ref[...] = jnp.zeros_like(sq2_ref)

    ids = ids_ref[0, 0, :]
    base, ovf = _block_window(ids, nseg)
    h = h_ref[...]
    if recompute:
        xphi = _mlp(h, fw[0][...], fw[1][...], fw[2][...], fw[3][...])
    else:
        xphi = xphi_ref[...]

    def finish(rho_b, scatter):
        h_new = h + ab_ref[0, :] * (xphi + rho_b) + ab_ref[1, :]
        hout_ref[...] = h_new
        xphi2 = _mlp(h_new, pw1_ref[...], pb1_ref[...],
                     pw2_ref[...], pb2_ref[...])
        if store_next:
            xphi2_ref[...] = xphi2
        sq = jnp.sum(xphi2 * xphi2, axis=0, keepdims=True)
        sq2_ref[...] += jnp.broadcast_to(sq, sq2_ref.shape)
        scatter(xphi2)

    @pl.when(jnp.logical_not(ovf))
    def _fast():
        oh = _onehot(ids - base, _W, F32)

        def scatter(xp2):
            ssp2_ref[pl.ds(base, _W), :] += _mmT(oh, xp2)

        finish(_mm(oh, rho_ref[pl.ds(base, _W), :]), scatter)

    @pl.when(ovf)
    def _slow():
        def scatter(xp2):
            for s0 in range(0, nseg, _CHUNK):
                ssp2_ref[s0:s0 + _CHUNK, :] += _mmT(
                    _onehot(ids - s0, _CHUNK, F32), xp2)

        finish(_gather_rho_slow(ids, nseg, rho_ref, h.shape), scatter)


def _fusedBPool_kernel(nseg, n_tok, ids_ref, h_ref, xphi_ref,
                       ssh_ref, ssp_ref, cnt_ref, sq_ref,
                       rw1_ref, rb1_ref, rw2_ref, rb2_ref, g_ref, bta_ref,
                       pw1_ref, pb1_ref, pw2_ref, pb2_ref,
                       ss_ref, rho_ref, ab_ref):
    i = pl.program_id(0)

    @pl.when(i == 0)
    def _prep():
        rho, a, b = _bn_affine(
            n_tok, ssh_ref[...], ssp_ref[...], cnt_ref[...], sq_ref[0, :],
            rw1_ref[...], rb1_ref[...], rw2_ref[...], rb2_ref[...],
            g_ref[0, :], bta_ref[0, :])
        rho_ref[...] = rho
        ab_ref[0, :] = a
        ab_ref[1, :] = b
        ss_ref[...] = jnp.zeros_like(ss_ref)

    ids = ids_ref[0, 0, :]
    base, ovf = _block_window(ids, nseg)
    h = h_ref[...]
    xphi = xphi_ref[...]

    def finish(rho_b, scatter):
        h_new = h + ab_ref[0, :] * (xphi + rho_b) + ab_ref[1, :]
        xp = _mlp(h_new, pw1_ref[...], pb1_ref[...],
                  pw2_ref[...], pb2_ref[...])
        scatter(xp)

    @pl.when(jnp.logical_not(ovf))
    def _fast():
        oh = _onehot(ids - base, _W, F32)

        def scatter(xp):
            ss_ref[pl.ds(base, _W), :] += _mmT(oh, xp)

        finish(_mm(oh, rho_ref[pl.ds(base, _W), :]), scatter)

    @pl.when(ovf)
    def _slow():
        def scatter(xp):
            for s0 in range(0, nseg, _CHUNK):
                ss_ref[s0:s0 + _CHUNK, :] += _mmT(
                    _onehot(ids - s0, _CHUNK, F32), xp)

        finish(_gather_rho_slow(ids, nseg, rho_ref, h.shape), scatter)


def _final_kernel(ss_ref, w1_ref, b1_ref, w2_ref, b2_ref, out_ref):
    out_ref[...] = _mlp(ss_ref[...], w1_ref[...], b1_ref[...],
                        w2_ref[...], b2_ref[...])


def _row(v):
    return v.reshape(1, -1)


def _mlp_args(p):
    return (p["l1"]["W"], _row(p["l1"]["b"]), p["l2"]["W"], _row(p["l2"]["b"]))


def _pick_block(n):
    for b in (1280, 1024, 640, 512, 256, 128, 64,
              32, 16, 8):
        if n % b == 0:
            return b
    return n


def kernel(x, segment_ids, params):
    n, d0 = x.shape
    nseg = _NSEG
    hdim = params["layers"][0]["phi"]["l2"]["W"].shape[1]
    b_tok = _pick_block(n)
    nb = n // b_tok
    ids3 = segment_ids.astype(jnp.int32).reshape(nb, 1, b_tok)

    cparams = pltpu.CompilerParams(dimension_semantics=("arbitrary",))
    small = pl.BlockSpec((1, hdim), lambda i: (0, 0))
    sqspec = pl.BlockSpec((8, hdim), lambda i: (0, 0))
    segspec = pl.BlockSpec((nseg, hdim), lambda i: (0, 0))
    idspec = pl.BlockSpec((1, 1, b_tok), lambda i: (i, 0, 0))
    wspec = pl.BlockSpec((hdim, hdim), lambda i: (0, 0))

    def tokspec(d):
        return pl.BlockSpec((b_tok, d), lambda i: (i, 0))

    seg_sds = jax.ShapeDtypeStruct((nseg, hdim), F32)
    tok_sds = jax.ShapeDtypeStruct((n, hdim), F32)
    sq_sds = jax.ShapeDtypeStruct((8, hdim), F32)
    mlp_specs = [wspec, small, wspec, small]

    passA = pl.pallas_call(
        functools.partial(_passA_kernel, nseg),
        grid=(nb,),
        in_specs=[idspec, tokspec(d0)] + mlp_specs,
        out_specs=[tokspec(hdim), segspec, segspec, segspec, sqspec],
        out_shape=[tok_sds, seg_sds, seg_sds, seg_sds, sq_sds],
        compiler_params=cparams,
    )

    stat_specs = [segspec, segspec, segspec, sqspec]
    scratch = [pltpu.VMEM((nseg, hdim), F32),
               pltpu.VMEM((8, hdim), F32)]

    def make_fusedBA(store_next, recompute):
        in_sp = [idspec, tokspec(hdim)]
        in_sp += mlp_specs if recompute else [tokspec(hdim)]
        in_sp += stat_specs + mlp_specs + [small, small] + mlp_specs
        out_sp = [tokspec(hdim)]
        out_sd = [tok_sds]
        if store_next:
            out_sp.append(tokspec(hdim))
            out_sd.append(tok_sds)
        out_sp += [segspec, segspec, sqspec]
        out_sd += [seg_sds, seg_sds, sq_sds]
        return pl.pallas_call(
            functools.partial(_fusedBA_kernel, nseg, float(n),
                              store_next, recompute),
            grid=(nb,),
            in_specs=in_sp,
            out_specs=out_sp,
            out_shape=out_sd,
            scratch_shapes=scratch,
            compiler_params=cparams,
        )

    fused_specs = ([idspec, tokspec(hdim), tokspec(hdim)]
                   + stat_specs + mlp_specs + [small, small] + mlp_specs)
    fusedBPool = pl.pallas_call(
        functools.partial(_fusedBPool_kernel, nseg, float(n)),
        grid=(nb,),
        in_specs=fused_specs,
        out_specs=segspec,
        out_shape=seg_sds,
        scratch_shapes=scratch,
        compiler_params=cparams,
    )

    layers = params["layers"]
    xphi, ssh, ssp, cnt, sq = passA(ids3, x, *_mlp_args(layers[0]["phi"]))
    h = x
    fBA_sr = make_fusedBA(store_next=False, recompute=False)
    fBA_rs = make_fusedBA(store_next=True, recompute=True)
    lp = layers[0]
    h, ssh, ssp, sq = fBA_sr(
        ids3, h, xphi, ssh, ssp, cnt, sq,
        *_mlp_args(lp["rho"]), _row(lp["bn"]["gamma"]),
        _row(lp["bn"]["beta"]), *_mlp_args(layers[1]["phi"]))
    lp = layers[1]
    h, xphi, ssh, ssp, sq = fBA_rs(
        ids3, h, *_mlp_args(layers[1]["phi"]), ssh, ssp, cnt, sq,
        *_mlp_args(lp["rho"]), _row(lp["bn"]["gamma"]),
        _row(lp["bn"]["beta"]), *_mlp_args(layers[2]["phi"]))

    lp = layers[-1]
    ss = fusedBPool(
        ids3, h, xphi, ssh, ssp, cnt, sq,
        *_mlp_args(lp["rho"]), _row(lp["bn"]["gamma"]),
        _row(lp["bn"]["beta"]), *_mlp_args(params["pool"]["phi"]))

    prho = params["pool"]["rho"]
    dout = prho["l2"]["W"].shape[1]
    out = pl.pallas_call(
        _final_kernel,
        in_specs=[pl.BlockSpec((nseg, hdim), lambda: (0, 0)),
                  pl.BlockSpec((hdim, hdim), lambda: (0, 0)),
                  pl.BlockSpec((1, hdim), lambda: (0, 0)),
                  pl.BlockSpec((hdim, dout), lambda: (0, 0)),
                  pl.BlockSpec((1, dout), lambda: (0, 0))],
        out_specs=pl.BlockSpec((nseg, dout), lambda: (0, 0)),
        out_shape=jax.ShapeDtypeStruct((nseg, dout), F32),
    )(ss, *_mlp_args(prho))
    return out


# speedup vs baseline: 1.7119x; 1.7119x over previous
import functools

import jax
import jax.numpy as jnp
from jax import lax
from jax.experimental import pallas as pl
from jax.experimental.pallas import tpu as pltpu

F32 = jnp.float32
_NSEG = 512
_W = 32
_CHUNK = 128
_EPS = 1e-5


def _mm(a, b):
    return jnp.dot(a, b, preferred_element_type=F32)


def _mmT(a, b):
    return lax.dot_general(a, b, (((0,), (0,)), ((), ())),
                           preferred_element_type=F32)


def _mlp(h, w1, b1, w2, b2):
    t = jnp.maximum(_mm(h, w1) + b1, 0.0)
    return _mm(t, w2) + b2


def _onehot(rel, w, dtype):
    b_tok = rel.shape[0]
    return (rel[:, None] == lax.broadcasted_iota(
        jnp.int32, (b_tok, w), 1)).astype(dtype)


def _block_window(ids, nseg):
    b_tok = ids.shape[0]
    base = (jnp.minimum(ids[0], nseg - _W) // 8) * 8
    ovf = (ids[b_tok - 1] - base) >= _W
    return base, ovf


def _bn_affine(n_tok, ssh, ssp, cnt, sq0, rw1, rb1, rw2, rb2, g, bta):
    rho = _mlp(ssh, rw1, rb1, rw2, rb2)
    s1 = jnp.sum(ssp, axis=0) + jnp.sum(cnt * rho, axis=0)
    s2 = (sq0 + 2.0 * jnp.sum(ssp * rho, axis=0)
          + jnp.sum(cnt * rho * rho, axis=0))
    m = s1 / n_tok
    v = s2 / n_tok - m * m
    a = g * lax.rsqrt(v + _EPS)
    return rho, a, bta - a * m


def _gather_rho_slow(ids, nseg, rho_ref, shape):
    acc = jnp.zeros(shape, F32)
    for s0 in range(0, nseg, _CHUNK):
        acc = acc + _mm(_onehot(ids - s0, _CHUNK, F32),
                        rho_ref[s0:s0 + _CHUNK, :])
    return acc


def _passA_kernel(nseg, ids_ref, h_ref, w1_ref, b1_ref, w2_ref, b2_ref,
                  xphi_ref, ssh_ref, ssp_ref, cnt_ref, sq_ref):
    i = pl.program_id(0)

    @pl.when(i == 0)
    def _init():
        ssh_ref[...] = jnp.zeros_like(ssh_ref)
        ssp_ref[...] = jnp.zeros_like(ssp_ref)
        cnt_ref[...] = jnp.zeros_like(cnt_ref)
        sq_ref[...] = jnp.zeros_like(sq_ref)

    h = h_ref[...]
    xphi = _mlp(h, w1_ref[...], b1_ref[...], w2_ref[...], b2_ref[...])
    xphi_ref[...] = xphi

    sq = jnp.sum(xphi * xphi, axis=0, keepdims=True)
    sq_ref[...] += jnp.broadcast_to(sq, sq_ref.shape)

    ids = ids_ref[0, 0, :]
    base, ovf = _block_window(ids, nseg)

    @pl.when(jnp.logical_not(ovf))
    def _fast():
        oh = _onehot(ids - base, _W, F32)
        ssh_ref[pl.ds(base, _W), :] += _mmT(oh, h)
        ssp_ref[pl.ds(base, _W), :] += _mmT(oh, xphi)
        c = jnp.sum(oh, axis=0, keepdims=True)
        cnt_ref[pl.ds(base, _W), :] += jnp.broadcast_to(
            c.T, (_W, cnt_ref.shape[1]))

    @pl.when(ovf)
    def _slow():
        for s0 in range(0, nseg, _CHUNK):
            ohc = _onehot(ids - s0, _CHUNK, F32)
            ssh_ref[s0:s0 + _CHUNK, :] += _mmT(ohc, h)
            ssp_ref[s0:s0 + _CHUNK, :] += _mmT(ohc, xphi)
            c = jnp.sum(ohc, axis=0, keepdims=True)
            cnt_ref[s0:s0 + _CHUNK, :] += jnp.broadcast_to(
                c.T, (_CHUNK, cnt_ref.shape[1]))


def _fusedBA_kernel(nseg, n_tok, store_next, recompute,
                    ids_ref, h_ref, *rest):
    rest = list(rest)
    fw = None
    if recompute:
        fw = [rest.pop(0) for _ in range(4)]
    else:
        xphi_ref = rest.pop(0)
    (ssh_ref, ssp_ref, cnt_ref, sq_ref,
     rw1_ref, rb1_ref, rw2_ref, rb2_ref, g_ref, bta_ref,
     pw1_ref, pb1_ref, pw2_ref, pb2_ref, hout_ref) = rest[:15]
    rest = rest[15:]
    if store_next:
        xphi2_ref = rest.pop(0)
    (ssh2_ref, ssp2_ref, sq2_ref, rho_ref, ab_ref) = rest

    i = pl.program_id(0)

    @pl.when(i == 0)
    def _prep():
        cnt = cnt_ref[...]
        rho, a, b = _bn_affine(
            n_tok, ssh_ref[...], ssp_ref[...], cnt, sq_ref[0, :],
            rw1_ref[...], rb1_ref[...], rw2_ref[...], rb2_ref[...],
            g_ref[0, :], bta_ref[0, :])
        rho_ref[...] = rho
        ab_ref[0, :] = a
        ab_ref[1, :] = b
        ssh2_ref[...] = (ssh_ref[...] + a * (ssp_ref[...] + cnt * rho)
                         + cnt * b)
        ssp2_ref[...] = jnp.zeros_like(ssp2_ref)
        sq2_ref[...] = jnp.zeros_like(sq2_ref)

    ids = ids_ref[0, 0, :]
    base, ovf = _block_window(ids, nseg)
    h = h_ref[...]
    if recompute:
        xphi = _mlp(h, fw[0][...], fw[1][...], fw[2][...], fw[3][...])
    else:
        xphi = xphi_ref[...]

    def finish(rho_b, scatter):
        h_new = h + ab_ref[0, :] * (xphi + rho_b) + ab_ref[1, :]
        hout_ref[...] = h_new
        xphi2 = _mlp(h_new, pw1_ref[...], pb1_ref[...],
                     pw2_ref[...], pb2_ref[...])
        if store_next:
            xphi2_ref[...] = xphi2
        sq = jnp.sum(xphi2 * xphi2, axis=0, keepdims=True)
        sq2_ref[...] += jnp.broadcast_to(sq, sq2_ref.shape)
        scatter(xphi2)

    @pl.when(jnp.logical_not(ovf))
    def _fast():
        oh = _onehot(ids - base, _W, F32)

        def scatter(xp2):
            ssp2_ref[pl.ds(base, _W), :] += _mmT(oh, xp2)

        finish(_mm(oh, rho_ref[pl.ds(base, _W), :]), scatter)

    @pl.when(ovf)
    def _slow():
        def scatter(xp2):
            for s0 in range(0, nseg, _CHUNK):
                ssp2_ref[s0:s0 + _CHUNK, :] += _mmT(
                    _onehot(ids - s0, _CHUNK, F32), xp2)

        finish(_gather_rho_slow(ids, nseg, rho_ref, h.shape), scatter)


def _fusedBPool_kernel(nseg, n_tok, ids_ref, h_ref, xphi_ref,
                       ssh_ref, ssp_ref, cnt_ref, sq_ref,
                       rw1_ref, rb1_ref, rw2_ref, rb2_ref, g_ref, bta_ref,
                       pw1_ref, pb1_ref, pw2_ref, pb2_ref,
                       ss_ref, rho_ref, ab_ref):
    i = pl.program_id(0)

    @pl.when(i == 0)
    def _prep():
        rho, a, b = _bn_affine(
            n_tok, ssh_ref[...], ssp_ref[...], cnt_ref[...], sq_ref[0, :],
            rw1_ref[...], rb1_ref[...], rw2_ref[...], rb2_ref[...],
            g_ref[0, :], bta_ref[0, :])
        rho_ref[...] = rho
        ab_ref[0, :] = a
        ab_ref[1, :] = b
        ss_ref[...] = jnp.zeros_like(ss_ref)

    ids = ids_ref[0, 0, :]
    base, ovf = _block_window(ids, nseg)
    h = h_ref[...]
    xphi = xphi_ref[...]

    def finish(rho_b, scatter):
        h_new = h + ab_ref[0, :] * (xphi + rho_b) + ab_ref[1, :]
        xp = _mlp(h_new, pw1_ref[...], pb1_ref[...],
                  pw2_ref[...], pb2_ref[...])
        scatter(xp)

    @pl.when(jnp.logical_not(ovf))
    def _fast():
        oh = _onehot(ids - base, _W, F32)

        def scatter(xp):
            ss_ref[pl.ds(base, _W), :] += _mmT(oh, xp)

        finish(_mm(oh, rho_ref[pl.ds(base, _W), :]), scatter)

    @pl.when(ovf)
    def _slow():
        def scatter(xp):
            for s0 in range(0, nseg, _CHUNK):
                ss_ref[s0:s0 + _CHUNK, :] += _mmT(
                    _onehot(ids - s0, _CHUNK, F32), xp)

        finish(_gather_rho_slow(ids, nseg, rho_ref, h.shape), scatter)


def _final_kernel(ss_ref, w1_ref, b1_ref, w2_ref, b2_ref, out_ref):
    out_ref[...] = _mlp(ss_ref[...], w1_ref[...], b1_ref[...],
                        w2_ref[...], b2_ref[...])


def _row(v):
    return v.reshape(1, -1)


def _mlp_args(p):
    return (p["l1"]["W"], _row(p["l1"]["b"]), p["l2"]["W"], _row(p["l2"]["b"]))


def _pick_block(n):
    for b in (4000, 3200, 2560, 2048, 2000, 1600, 1280, 1024, 640, 512,
              256, 128, 64, 32, 16, 8):
        if n % b == 0:
            return b
    return n


def kernel(x, segment_ids, params):
    n, d0 = x.shape
    nseg = _NSEG
    hdim = params["layers"][0]["phi"]["l2"]["W"].shape[1]
    b_tok = _pick_block(n)
    nb = n // b_tok
    ids3 = segment_ids.astype(jnp.int32).reshape(nb, 1, b_tok)

    cparams = pltpu.CompilerParams(dimension_semantics=("arbitrary",))
    small = pl.BlockSpec((1, hdim), lambda i: (0, 0))
    sqspec = pl.BlockSpec((8, hdim), lambda i: (0, 0))
    segspec = pl.BlockSpec((nseg, hdim), lambda i: (0, 0))
    idspec = pl.BlockSpec((1, 1, b_tok), lambda i: (i, 0, 0))
    wspec = pl.BlockSpec((hdim, hdim), lambda i: (0, 0))

    def tokspec(d):
        return pl.BlockSpec((b_tok, d), lambda i: (i, 0))

    seg_sds = jax.ShapeDtypeStruct((nseg, hdim), F32)
    tok_sds = jax.ShapeDtypeStruct((n, hdim), F32)
    sq_sds = jax.ShapeDtypeStruct((8, hdim), F32)
    mlp_specs = [wspec, small, wspec, small]

    passA = pl.pallas_call(
        functools.partial(_passA_kernel, nseg),
        grid=(nb,),
        in_specs=[idspec, tokspec(d0)] + mlp_specs,
        out_specs=[tokspec(hdim), segspec, segspec, segspec, sqspec],
        out_shape=[tok_sds, seg_sds, seg_sds, seg_sds, sq_sds],
        compiler_params=cparams,
    )

    stat_specs = [segspec, segspec, segspec, sqspec]
    scratch = [pltpu.VMEM((nseg, hdim), F32),
               pltpu.VMEM((8, hdim), F32)]

    def make_fusedBA(store_next, recompute):
        in_sp = [idspec, tokspec(hdim)]
        in_sp += mlp_specs if recompute else [tokspec(hdim)]
        in_sp += stat_specs + mlp_specs + [small, small] + mlp_specs
        out_sp = [tokspec(hdim)]
        out_sd = [tok_sds]
        if store_next:
            out_sp.append(tokspec(hdim))
            out_sd.append(tok_sds)
        out_sp += [segspec, segspec, sqspec]
        out_sd += [seg_sds, seg_sds, sq_sds]
        return pl.pallas_call(
            functools.partial(_fusedBA_kernel, nseg, float(n),
                              store_next, recompute),
            grid=(nb,),
            in_specs=in_sp,
            out_specs=out_sp,
            out_shape=out_sd,
            scratch_shapes=scratch,
            compiler_params=cparams,
        )

    fused_specs = ([idspec, tokspec(hdim), tokspec(hdim)]
                   + stat_specs + mlp_specs + [small, small] + mlp_specs)
    fusedBPool = pl.pallas_call(
        functools.partial(_fusedBPool_kernel, nseg, float(n)),
        grid=(nb,),
        in_specs=fused_specs,
        out_specs=segspec,
        out_shape=seg_sds,
        scratch_shapes=scratch,
        compiler_params=cparams,
    )

    layers = params["layers"]
    xphi, ssh, ssp, cnt, sq = passA(ids3, x, *_mlp_args(layers[0]["phi"]))
    h = x
    fBA_sr = make_fusedBA(store_next=False, recompute=False)
    fBA_rs = make_fusedBA(store_next=True, recompute=True)
    lp = layers[0]
    h, ssh, ssp, sq = fBA_sr(
        ids3, h, xphi, ssh, ssp, cnt, sq,
        *_mlp_args(lp["rho"]), _row(lp["bn"]["gamma"]),
        _row(lp["bn"]["beta"]), *_mlp_args(layers[1]["phi"]))
    lp = layers[1]
    h, xphi, ssh, ssp, sq = fBA_rs(
        ids3, h, *_mlp_args(layers[1]["phi"]), ssh, ssp, cnt, sq,
        *_mlp_args(lp["rho"]), _row(lp["bn"]["gamma"]),
        _row(lp["bn"]["beta"]), *_mlp_args(layers[2]["phi"]))

    lp = layers[-1]
    ss = fusedBPool(
        ids3, h, xphi, ssh, ssp, cnt, sq,
        *_mlp_args(lp["rho"]), _row(lp["bn"]["gamma"]),
        _row(lp["bn"]["beta"]), *_mlp_args(params["pool"]["phi"]))

    prho = params["pool"]["rho"]
    dout = prho["l2"]["W"].shape[1]
    out = pl.pallas_call(
        _final_kernel,
        in_specs=[pl.BlockSpec((nseg, hdim), lambda: (0, 0)),
                  pl.BlockSpec((hdim, hdim), lambda: (0, 0)),
                  pl.BlockSpec((1, hdim), lambda: (0, 0)),
                  pl.BlockSpec((hdim, dout), lambda: (0, 0)),
                  pl.BlockSpec((1, dout), lambda: (0, 0))],
        out_specs=pl.BlockSpec((nseg, dout), lambda: (0, 0)),
        out_shape=jax.ShapeDtypeStruct((nseg, dout), F32),
    )(ss, *_mlp_args(prho))
    return out


# speedup vs baseline: 1.9741x; 1.1532x over previous
import functools

import jax
import jax.numpy as jnp
from jax import lax
from jax.experimental import pallas as pl
from jax.experimental.pallas import tpu as pltpu

F32 = jnp.float32
_NSEG = 512
_W = 32
_CHUNK = 128
_EPS = 1e-5


def _mm(a, b):
    return jnp.dot(a, b, preferred_element_type=F32)


def _mmT(a, b):
    return lax.dot_general(a, b, (((0,), (0,)), ((), ())),
                           preferred_element_type=F32)


def _mlp(h, w1, b1, w2, b2):
    t = jnp.maximum(_mm(h, w1) + b1, 0.0)
    return _mm(t, w2) + b2


def _onehot(rel, w, dtype):
    b_tok = rel.shape[0]
    return (rel[:, None] == lax.broadcasted_iota(
        jnp.int32, (b_tok, w), 1)).astype(dtype)


def _block_window(ids, nseg):
    b_tok = ids.shape[0]
    base = (jnp.minimum(ids[0], nseg - _W) // 8) * 8
    ovf = (ids[b_tok - 1] - base) >= _W
    return base, ovf


def _bn_affine(n_tok, ssh, ssp, cnt, sq0, rw1, rb1, rw2, rb2, g, bta):
    rho = _mlp(ssh, rw1, rb1, rw2, rb2)
    s1 = jnp.sum(ssp, axis=0) + jnp.sum(cnt * rho, axis=0)
    s2 = (sq0 + 2.0 * jnp.sum(ssp * rho, axis=0)
          + jnp.sum(cnt * rho * rho, axis=0))
    m = s1 / n_tok
    v = s2 / n_tok - m * m
    a = g * lax.rsqrt(v + _EPS)
    return rho, a, bta - a * m


def _gather_rho_slow(ids, nseg, rho_ref, shape):
    acc = jnp.zeros(shape, F32)
    for s0 in range(0, nseg, _CHUNK):
        acc = acc + _mm(_onehot(ids - s0, _CHUNK, F32),
                        rho_ref[s0:s0 + _CHUNK, :])
    return acc


def _passA_kernel(nseg, ids_ref, h_ref, w1_ref, b1_ref, w2_ref, b2_ref,
                  xphi_ref, ssh_ref, ssp_ref, cnt_ref, sq_ref):
    i = pl.program_id(0)

    @pl.when(i == 0)
    def _init():
        ssh_ref[...] = jnp.zeros_like(ssh_ref)
        ssp_ref[...] = jnp.zeros_like(ssp_ref)
        cnt_ref[...] = jnp.zeros_like(cnt_ref)
        sq_ref[...] = jnp.zeros_like(sq_ref)

    h = h_ref[...]
    xphi = _mlp(h, w1_ref[...], b1_ref[...], w2_ref[...], b2_ref[...])
    xphi_ref[...] = xphi

    sq = jnp.sum(xphi * xphi, axis=0, keepdims=True)
    sq_ref[...] += jnp.broadcast_to(sq, sq_ref.shape)

    ids = ids_ref[0, 0, :]
    base, ovf = _block_window(ids, nseg)

    @pl.when(jnp.logical_not(ovf))
    def _fast():
        oh = _onehot(ids - base, _W, F32)
        ssh_ref[pl.ds(base, _W), :] += _mmT(oh, h)
        ssp_ref[pl.ds(base, _W), :] += _mmT(oh, xphi)
        c = jnp.sum(oh, axis=0, keepdims=True)
        cnt_ref[pl.ds(base, _W), :] += jnp.broadcast_to(
            c.T, (_W, cnt_ref.shape[1]))

    @pl.when(ovf)
    def _slow():
        for s0 in range(0, nseg, _CHUNK):
            ohc = _onehot(ids - s0, _CHUNK, F32)
            ssh_ref[s0:s0 + _CHUNK, :] += _mmT(ohc, h)
            ssp_ref[s0:s0 + _CHUNK, :] += _mmT(ohc, xphi)
            c = jnp.sum(ohc, axis=0, keepdims=True)
            cnt_ref[s0:s0 + _CHUNK, :] += jnp.broadcast_to(
                c.T, (_CHUNK, cnt_ref.shape[1]))


def _fusedBA_kernel(nseg, n_tok, store_next, recompute,
                    ids_ref, h_ref, *rest):
    rest = list(rest)
    fw = None
    if recompute:
        fw = [rest.pop(0) for _ in range(4)]
    else:
        xphi_ref = rest.pop(0)
    (ssh_ref, ssp_ref, cnt_ref, sq_ref,
     rw1_ref, rb1_ref, rw2_ref, rb2_ref, g_ref, bta_ref,
     pw1_ref, pb1_ref, pw2_ref, pb2_ref, hout_ref) = rest[:15]
    rest = rest[15:]
    if store_next:
        xphi2_ref = rest.pop(0)
    (ssh2_ref, ssp2_ref, sq2_ref, rho_ref, ab_ref) = rest

    i = pl.program_id(0)

    @pl.when(i == 0)
    def _prep():
        cnt = cnt_ref[...]
        rho, a, b = _bn_affine(
            n_tok, ssh_ref[...], ssp_ref[...], cnt, sq_ref[0, :],
            rw1_ref[...], rb1_ref[...], rw2_ref[...], rb2_ref[...],
            g_ref[0, :], bta_ref[0, :])
        rho_ref[...] = rho
        ab_ref[0, :] = a
        ab_ref[1, :] = b
        ssh2_ref[...] = (ssh_ref[...] + a * (ssp_ref[...] + cnt * rho)
                         + cnt * b)
        ssp2_ref[...] = jnp.zeros_like(ssp2_ref)
        sq2_ref[...] = jnp.zeros_like(sq2_ref)

    ids = ids_ref[0, 0, :]
    base, ovf = _block_window(ids, nseg)
    h = h_ref[...]
    if recompute:
        xphi = _mlp(h, fw[0][...], fw[1][...], fw[2][...], fw[3][...])
    else:
        xphi = xphi_ref[...]

    def finish(rho_b, scatter):
        h_new = h + ab_ref[0, :] * (xphi + rho_b) + ab_ref[1, :]
        hout_ref[...] = h_new
        xphi2 = _mlp(h_new, pw1_ref[...], pb1_ref[...],
                     pw2_ref[...], pb2_ref[...])
        if store_next:
            xphi2_ref[...] = xphi2
        sq = jnp.sum(xphi2 * xphi2, axis=0, keepdims=True)
        sq2_ref[...] += jnp.broadcast_to(sq, sq2_ref.shape)
        scatter(xphi2)

    @pl.when(jnp.logical_not(ovf))
    def _fast():
        oh = _onehot(ids - base, _W, F32)

        def scatter(xp2):
            ssp2_ref[pl.ds(base, _W), :] += _mmT(oh, xp2)

        finish(_mm(oh, rho_ref[pl.ds(base, _W), :]), scatter)

    @pl.when(ovf)
    def _slow():
        def scatter(xp2):
            for s0 in range(0, nseg, _CHUNK):
                ssp2_ref[s0:s0 + _CHUNK, :] += _mmT(
                    _onehot(ids - s0, _CHUNK, F32), xp2)

        finish(_gather_rho_slow(ids, nseg, rho_ref, h.shape), scatter)


def _fusedBPool_kernel(nseg, n_tok, ids_ref, h_ref, xphi_ref,
                       ssh_ref, ssp_ref, cnt_ref, sq_ref,
                       rw1_ref, rb1_ref, rw2_ref, rb2_ref, g_ref, bta_ref,
                       pw1_ref, pb1_ref, pw2_ref, pb2_ref,
                       ss_ref, rho_ref, ab_ref):
    i = pl.program_id(0)

    @pl.when(i == 0)
    def _prep():
        rho, a, b = _bn_affine(
            n_tok, ssh_ref[...], ssp_ref[...], cnt_ref[...], sq_ref[0, :],
            rw1_ref[...], rb1_ref[...], rw2_ref[...], rb2_ref[...],
            g_ref[0, :], bta_ref[0, :])
        rho_ref[...] = rho
        ab_ref[0, :] = a
        ab_ref[1, :] = b
        ss_ref[...] = jnp.zeros_like(ss_ref)

    ids = ids_ref[0, 0, :]
    base, ovf = _block_window(ids, nseg)
    h = h_ref[...]
    xphi = xphi_ref[...]

    def finish(rho_b, scatter):
        h_new = h + ab_ref[0, :] * (xphi + rho_b) + ab_ref[1, :]
        xp = _mlp(h_new, pw1_ref[...], pb1_ref[...],
                  pw2_ref[...], pb2_ref[...])
        scatter(xp)

    @pl.when(jnp.logical_not(ovf))
    def _fast():
        oh = _onehot(ids - base, _W, F32)

        def scatter(xp):
            ss_ref[pl.ds(base, _W), :] += _mmT(oh, xp)

        finish(_mm(oh, rho_ref[pl.ds(base, _W), :]), scatter)

    @pl.when(ovf)
    def _slow():
        def scatter(xp):
            for s0 in range(0, nseg, _CHUNK):
                ss_ref[s0:s0 + _CHUNK, :] += _mmT(
                    _onehot(ids - s0, _CHUNK, F32), xp)

        finish(_gather_rho_slow(ids, nseg, rho_ref, h.shape), scatter)


def _final_kernel(ss_ref, w1_ref, b1_ref, w2_ref, b2_ref, out_ref):
    out_ref[...] = _mlp(ss_ref[...], w1_ref[...], b1_ref[...],
                        w2_ref[...], b2_ref[...])


def _row(v):
    return v.reshape(1, -1)


def _mlp_args(p):
    return (p["l1"]["W"], _row(p["l1"]["b"]), p["l2"]["W"], _row(p["l2"]["b"]))


def _pick_block(n):
    for b in (8000, 6400, 5000, 4000, 3200, 2560, 2048, 2000, 1600, 1280,
              1024, 640, 512, 256, 128, 64, 32, 16, 8):
        if n % b == 0:
            return b
    return n


def kernel(x, segment_ids, params):
    n, d0 = x.shape
    nseg = _NSEG
    hdim = params["layers"][0]["phi"]["l2"]["W"].shape[1]
    b_tok = _pick_block(n)
    nb = n // b_tok
    ids3 = segment_ids.astype(jnp.int32).reshape(nb, 1, b_tok)

    cparams = pltpu.CompilerParams(dimension_semantics=("arbitrary",))
    small = pl.BlockSpec((1, hdim), lambda i: (0, 0))
    sqspec = pl.BlockSpec((8, hdim), lambda i: (0, 0))
    segspec = pl.BlockSpec((nseg, hdim), lambda i: (0, 0))
    idspec = pl.BlockSpec((1, 1, b_tok), lambda i: (i, 0, 0))
    wspec = pl.BlockSpec((hdim, hdim), lambda i: (0, 0))

    def tokspec(d):
        return pl.BlockSpec((b_tok, d), lambda i: (i, 0))

    seg_sds = jax.ShapeDtypeStruct((nseg, hdim), F32)
    tok_sds = jax.ShapeDtypeStruct((n, hdim), F32)
    sq_sds = jax.ShapeDtypeStruct((8, hdim), F32)
    mlp_specs = [wspec, small, wspec, small]

    passA = pl.pallas_call(
        functools.partial(_passA_kernel, nseg),
        grid=(nb,),
        in_specs=[idspec, tokspec(d0)] + mlp_specs,
        out_specs=[tokspec(hdim), segspec, segspec, segspec, sqspec],
        out_shape=[tok_sds, seg_sds, seg_sds, seg_sds, sq_sds],
        compiler_params=cparams,
    )

    stat_specs = [segspec, segspec, segspec, sqspec]
    scratch = [pltpu.VMEM((nseg, hdim), F32),
               pltpu.VMEM((8, hdim), F32)]

    def make_fusedBA(store_next, recompute):
        in_sp = [idspec, tokspec(hdim)]
        in_sp += mlp_specs if recompute else [tokspec(hdim)]
        in_sp += stat_specs + mlp_specs + [small, small] + mlp_specs
        out_sp = [tokspec(hdim)]
        out_sd = [tok_sds]
        if store_next:
            out_sp.append(tokspec(hdim))
            out_sd.append(tok_sds)
        out_sp += [segspec, segspec, sqspec]
        out_sd += [seg_sds, seg_sds, sq_sds]
        return pl.pallas_call(
            functools.partial(_fusedBA_kernel, nseg, float(n),
                              store_next, recompute),
            grid=(nb,),
            in_specs=in_sp,
            out_specs=out_sp,
            out_shape=out_sd,
            scratch_shapes=scratch,
            compiler_params=cparams,
        )

    fused_specs = ([idspec, tokspec(hdim), tokspec(hdim)]
                   + stat_specs + mlp_specs + [small, small] + mlp_specs)
    fusedBPool = pl.pallas_call(
        functools.partial(_fusedBPool_kernel, nseg, float(n)),
        grid=(nb,),
        in_specs=fused_specs,
        out_specs=segspec,
        out_shape=seg_sds,
        scratch_shapes=scratch,
        compiler_params=cparams,
    )

    layers = params["layers"]
    xphi, ssh, ssp, cnt, sq = passA(ids3, x, *_mlp_args(layers[0]["phi"]))
    h = x
    fBA_sr = make_fusedBA(store_next=False, recompute=False)
    fBA_rs = make_fusedBA(store_next=True, recompute=True)
    lp = layers[0]
    h, ssh, ssp, sq = fBA_sr(
        ids3, h, xphi, ssh, ssp, cnt, sq,
        *_mlp_args(lp["rho"]), _row(lp["bn"]["gamma"]),
        _row(lp["bn"]["beta"]), *_mlp_args(layers[1]["phi"]))
    lp = layers[1]
    h, xphi, ssh, ssp, sq = fBA_rs(
        ids3, h, *_mlp_args(layers[1]["phi"]), ssh, ssp, cnt, sq,
        *_mlp_args(lp["rho"]), _row(lp["bn"]["gamma"]),
        _row(lp["bn"]["beta"]), *_mlp_args(layers[2]["phi"]))

    lp = layers[-1]
    ss = fusedBPool(
        ids3, h, xphi, ssh, ssp, cnt, sq,
        *_mlp_args(lp["rho"]), _row(lp["bn"]["gamma"]),
        _row(lp["bn"]["beta"]), *_mlp_args(params["pool"]["phi"]))

    prho = params["pool"]["rho"]
    dout = prho["l2"]["W"].shape[1]
    out = pl.pallas_call(
        _final_kernel,
        in_specs=[pl.BlockSpec((nseg, hdim), lambda: (0, 0)),
                  pl.BlockSpec((hdim, hdim), lambda: (0, 0)),
                  pl.BlockSpec((1, hdim), lambda: (0, 0)),
                  pl.BlockSpec((hdim, dout), lambda: (0, 0)),
                  pl.BlockSpec((1, dout), lambda: (0, 0))],
        out_specs=pl.BlockSpec((nseg, dout), lambda: (0, 0)),
        out_shape=jax.ShapeDtypeStruct((nseg, dout), F32),
    )(ss, *_mlp_args(prho))
    return out
